# baseline (speedup 1.0000x reference)
import functools

import jax
import jax.numpy as jnp
from jax.experimental import pallas as pl
from jax.experimental.pallas import tpu as pltpu

_EPS = 1e-12


def _body(g_ref, row_ref, col_ref, tte_ref, w_ref, b_ref, out_ref, acc_ref,
          *, num_frames):
    f = pl.program_id(0)

    @pl.when(f == 0)
    def _init():
        acc_ref[...] = g_ref[0]

    @pl.when(f > 0)
    def _accum():
        acc_ref[...] += g_ref[0]

    @pl.when(f == num_frames - 1)
    def _finish():
        m = acc_ref[...] * (1.0 / num_frames)
        emb = (m + row_ref[...][:, None, :] + col_ref[...][None, :, :]
               + tte_ref[...][None, :, :])
        mu = jnp.mean(emb, axis=-1, keepdims=True)
        d = emb - mu
        var = jnp.mean(d * d, axis=-1, keepdims=True)
        y = d * jax.lax.rsqrt(var + _EPS)
        out_ref[...] = y * w_ref[...][None, None, :] + b_ref[...][None, None, :]


def kernel(grid, row_emb, col_emb, token_type_emb, ln_weight, ln_bias):
    B, F, H, W, C = grid.shape
    g = grid.reshape(F, H, W, C)

    out = pl.pallas_call(
        functools.partial(_body, num_frames=F),
        grid=(F,),
        in_specs=[
            pl.BlockSpec((1, H, W, C), lambda f: (f, 0, 0, 0)),
            pl.BlockSpec((H, C), lambda f: (0, 0)),
            pl.BlockSpec((W, C), lambda f: (0, 0)),
            pl.BlockSpec((1, C), lambda f: (0, 0)),
            pl.BlockSpec((C,), lambda f: (0,)),
            pl.BlockSpec((C,), lambda f: (0,)),
        ],
        out_specs=pl.BlockSpec((H, W, C), lambda f: (0, 0, 0)),
        out_shape=jax.ShapeDtypeStruct((H, W, C), jnp.float32),
        scratch_shapes=[pltpu.VMEM((H, W, C), jnp.float32)],
        compiler_params=pltpu.CompilerParams(
            dimension_semantics=("arbitrary",),
        ),
    )(g, row_emb, col_emb, token_type_emb, ln_weight, ln_bias)

    return out.reshape(B, H * W, C)


# speedup vs baseline: 1.2679x; 1.2679x over previous
import functools

import jax
import jax.numpy as jnp
from jax.experimental import pallas as pl
from jax.experimental.pallas import tpu as pltpu

_EPS = 1e-12


_FB = 4


def _body(g_ref, row_ref, col_ref, tte_ref, w_ref, b_ref, out_ref, acc_ref,
          *, num_steps, num_frames):
    f = pl.program_id(0)
    s = ((g_ref[0] + g_ref[1]) + (g_ref[2] + g_ref[3]))

    @pl.when(f == 0)
    def _init():
        acc_ref[...] = s

    @pl.when(f > 0)
    def _accum():
        acc_ref[...] += s

    @pl.when(f == num_steps - 1)
    def _finish():
        m = acc_ref[...] * (1.0 / num_frames)
        emb = (m + row_ref[...][:, None, :] + col_ref[...][None, :, :]
               + tte_ref[...][None, :, :])
        mu = jnp.mean(emb, axis=-1, keepdims=True)
        d = emb - mu
        var = jnp.mean(d * d, axis=-1, keepdims=True)
        y = d * jax.lax.rsqrt(var + _EPS)
        out_ref[...] = y * w_ref[...][None, None, :] + b_ref[...][None, None, :]


def kernel(grid, row_emb, col_emb, token_type_emb, ln_weight, ln_bias):
    B, F, H, W, C = grid.shape
    g = grid.reshape(F, H, W, C)

    out = pl.pallas_call(
        functools.partial(_body, num_steps=F // _FB, num_frames=F),
        grid=(F // _FB,),
        in_specs=[
            pl.BlockSpec((_FB, H, W, C), lambda f: (f, 0, 0, 0)),
            pl.BlockSpec((H, C), lambda f: (0, 0)),
            pl.BlockSpec((W, C), lambda f: (0, 0)),
            pl.BlockSpec((1, C), lambda f: (0, 0)),
            pl.BlockSpec((C,), lambda f: (0,)),
            pl.BlockSpec((C,), lambda f: (0,)),
        ],
        out_specs=pl.BlockSpec((H, W, C), lambda f: (0, 0, 0)),
        out_shape=jax.ShapeDtypeStruct((H, W, C), jnp.float32),
        scratch_shapes=[pltpu.VMEM((H, W, C), jnp.float32)],
        compiler_params=pltpu.CompilerParams(
            dimension_semantics=("arbitrary",),
        ),
    )(g, row_emb, col_emb, token_type_emb, ln_weight, ln_bias)

    return out.reshape(B, H * W, C)
